# Initial kernel scaffold; baseline (speedup 1.0000x reference)
import jax
import jax.numpy as jnp
from jax.experimental import pallas as pl


def kernel(x, fc_weight, bias):
    raise NotImplementedError("write your pallas kernel here")


# speedup vs baseline: 1.2186x; 1.2186x over previous
import functools

import jax
import jax.numpy as jnp
from jax import lax
from jax.experimental import pallas as pl
from jax.experimental.pallas import tpu as pltpu
from jax.experimental.pallas import tpu_sc as plsc

NUM_FIELDS = 26
BATCH = 16384
FIELD_SIZE = 100000
LANES = 16
NUM_WORKERS = 32
BPW = BATCH // NUM_WORKERS
JV = BPW // LANES
N_EL = NUM_FIELDS * BPW
NCHUNK = N_EL // LANES

_mesh = plsc.VectorSubcoreMesh(core_axis_name="c", subcore_axis_name="s")


@functools.partial(
    pl.kernel,
    mesh=_mesh,
    out_type=jax.ShapeDtypeStruct((BATCH,), jnp.float32),
    scratch_types=[
        pltpu.VMEM((N_EL,), jnp.int32),
        pltpu.VMEM((N_EL,), jnp.float32),
        pltpu.VMEM((LANES,), jnp.float32),
        pltpu.VMEM((BPW,), jnp.float32),
        pltpu.SemaphoreType.DMA,
    ],
)
def _emb_sum_kernel(x_hbm, w_hbm, b_hbm, out_hbm, xv, gath, bias_v, out_v,
                    sem):
    wid = lax.axis_index("s") * 2 + lax.axis_index("c")
    base = wid * BPW
    pltpu.sync_copy(x_hbm.at[pl.ds(wid * N_EL, N_EL)], xv)
    pltpu.sync_copy(b_hbm, bias_v)

    def off_body(v, _):
        off = (v // JV) * FIELD_SIZE
        xv[pl.ds(v * LANES, LANES)] = xv[pl.ds(v * LANES, LANES)] + off
        return 0

    lax.fori_loop(0, NCHUNK, off_body, 0)

    pltpu.async_copy(w_hbm.at[xv], gath, sem).wait()

    bvec = bias_v[...]

    def r_body(j, _):
        def a_body(f, acc):
            return acc + gath[pl.ds(f * BPW + j * LANES, LANES)]

        out_v[pl.ds(j * LANES, LANES)] = lax.fori_loop(0, NUM_FIELDS, a_body,
                                                       bvec)
        return 0

    lax.fori_loop(0, JV, r_body, 0)
    pltpu.sync_copy(out_v, out_hbm.at[pl.ds(base, BPW)])


def kernel(x, fc_weight, bias):
    xp = (x.astype(jnp.int32).T
          .reshape(NUM_FIELDS, NUM_WORKERS, BPW)
          .transpose(1, 0, 2)
          .reshape(-1))
    w_flat = fc_weight.reshape(-1)
    b16 = jnp.broadcast_to(bias.astype(jnp.float32), (LANES,))
    out = _emb_sum_kernel(xp, w_flat, b16)
    return out.reshape(BATCH, 1)


# speedup vs baseline: 1.9944x; 1.6366x over previous
import functools

import jax
import jax.numpy as jnp
from jax import lax
from jax.experimental import pallas as pl
from jax.experimental.pallas import tpu as pltpu
from jax.experimental.pallas import tpu_sc as plsc

NUM_FIELDS = 26
BATCH = 16384
FIELD_SIZE = 100000
LANES = 16
NUM_WORKERS = 32
BPW = BATCH // NUM_WORKERS
JV = BPW // LANES
N_EL = NUM_FIELDS * BPW

CHUNKS = (7, 7, 6, 6)

_mesh = plsc.VectorSubcoreMesh(core_axis_name="c", subcore_axis_name="s")


def _make_chunk_kernel(f0, nf, add_bias):
    n_el = nf * BPW

    @functools.partial(
        pl.kernel,
        mesh=_mesh,
        out_type=jax.ShapeDtypeStruct((BATCH,), jnp.float32),
        scratch_types=[
            pltpu.VMEM((n_el,), jnp.int32),
            pltpu.VMEM((n_el,), jnp.float32),
            pltpu.VMEM((LANES,), jnp.float32),
            pltpu.VMEM((BPW,), jnp.float32),
            pltpu.SemaphoreType.DMA,
        ],
    )
    def _chunk_kernel(x_hbm, w_hbm, b_hbm, out_hbm, xv, gath, bias_v, out_v,
                      sem):
        wid = lax.axis_index("s") * 2 + lax.axis_index("c")
        base = wid * BPW
        pltpu.sync_copy(x_hbm.at[pl.ds(wid * N_EL + f0 * BPW, n_el)], xv)
        if add_bias:
            pltpu.sync_copy(b_hbm, bias_v)

        def off_body(v, _):
            off = (v // JV) * FIELD_SIZE
            xv[pl.ds(v * LANES, LANES)] = xv[pl.ds(v * LANES, LANES)] + off
            return 0

        lax.fori_loop(0, nf * JV, off_body, 0)

        pltpu.async_copy(w_hbm.at[xv], gath, sem).wait()

        if add_bias:
            seed = bias_v[...]
        else:
            seed = jnp.zeros((LANES,), jnp.float32)

        def r_body(j, _):
            def a_body(f, acc):
                return acc + gath[pl.ds(f * BPW + j * LANES, LANES)]

            out_v[pl.ds(j * LANES, LANES)] = lax.fori_loop(0, nf, a_body,
                                                           seed)
            return 0

        lax.fori_loop(0, JV, r_body, 0)
        pltpu.sync_copy(out_v, out_hbm.at[pl.ds(base, BPW)])

    return _chunk_kernel


_F0S = [sum(CHUNKS[:k]) for k in range(len(CHUNKS))]
_CHUNK_KERNELS = [
    _make_chunk_kernel(f0, nf, f0 == 0) for f0, nf in zip(_F0S, CHUNKS)
]


def kernel(x, fc_weight, bias):
    xp = (x.astype(jnp.int32).T
          .reshape(NUM_FIELDS, NUM_WORKERS, BPW)
          .transpose(1, 0, 2)
          .reshape(-1))
    b16 = jnp.broadcast_to(bias.astype(jnp.float32), (LANES,))
    total = None
    for f0, nf, ck in zip(_F0S, CHUNKS, _CHUNK_KERNELS):
        w_k = fc_weight[f0 * FIELD_SIZE:(f0 + nf) * FIELD_SIZE].reshape(-1)
        part = ck(xp, w_k, b16)
        total = part if total is None else total + part
    return total.reshape(BATCH, 1)


# speedup vs baseline: 2.3618x; 1.1842x over previous
import functools

import jax
import jax.numpy as jnp
from jax import lax
from jax.experimental import pallas as pl
from jax.experimental.pallas import tpu as pltpu
from jax.experimental.pallas import tpu_sc as plsc

NUM_FIELDS = 26
BATCH = 16384
FIELD_SIZE = 100000
LANES = 16
NUM_WORKERS = 32
BPW = BATCH // NUM_WORKERS
JV = BPW // LANES
N_EL = NUM_FIELDS * BPW

CHUNKS = (2, 6, 6, 6, 6)

_mesh = plsc.VectorSubcoreMesh(core_axis_name="c", subcore_axis_name="s")


def _make_chunk_kernel(f0, nf, add_bias):
    n_el = nf * BPW

    @functools.partial(
        pl.kernel,
        mesh=_mesh,
        out_type=jax.ShapeDtypeStruct((BATCH,), jnp.float32),
        scratch_types=[
            pltpu.VMEM((n_el,), jnp.int32),
            pltpu.VMEM((n_el,), jnp.float32),
            pltpu.VMEM((LANES,), jnp.float32),
            pltpu.VMEM((BPW,), jnp.float32),
            pltpu.SemaphoreType.DMA,
        ],
    )
    def _chunk_kernel(x_hbm, w_hbm, b_hbm, out_hbm, xv, gath, bias_v, out_v,
                      sem):
        wid = lax.axis_index("s") * 2 + lax.axis_index("c")
        base = wid * BPW
        pltpu.sync_copy(x_hbm.at[pl.ds(wid * N_EL + f0 * BPW, n_el)], xv)
        if add_bias:
            pltpu.sync_copy(b_hbm, bias_v)

        def off_body(v, _):
            off = (v // JV) * FIELD_SIZE
            xv[pl.ds(v * LANES, LANES)] = xv[pl.ds(v * LANES, LANES)] + off
            return 0

        lax.fori_loop(0, nf * JV, off_body, 0)

        pltpu.async_copy(w_hbm.at[xv], gath, sem).wait()

        if add_bias:
            seed = bias_v[...]
        else:
            seed = jnp.zeros((LANES,), jnp.float32)

        def r_body(j, _):
            def a_body(f, acc):
                return acc + gath[pl.ds(f * BPW + j * LANES, LANES)]

            out_v[pl.ds(j * LANES, LANES)] = lax.fori_loop(0, nf, a_body,
                                                           seed)
            return 0

        lax.fori_loop(0, JV, r_body, 0)
        pltpu.sync_copy(out_v, out_hbm.at[pl.ds(base, BPW)])

    return _chunk_kernel


_F0S = [sum(CHUNKS[:k]) for k in range(len(CHUNKS))]
_CHUNK_KERNELS = [
    _make_chunk_kernel(f0, nf, f0 == 0) for f0, nf in zip(_F0S, CHUNKS)
]


def kernel(x, fc_weight, bias):
    xp = (x.astype(jnp.int32).T
          .reshape(NUM_FIELDS, NUM_WORKERS, BPW)
          .transpose(1, 0, 2)
          .reshape(-1))
    b16 = jnp.broadcast_to(bias.astype(jnp.float32), (LANES,))
    total = None
    for f0, nf, ck in zip(_F0S, CHUNKS, _CHUNK_KERNELS):
        w_k2 = lax.slice(fc_weight, (f0 * FIELD_SIZE, 0),
                         ((f0 + nf) * FIELD_SIZE, 1))
        w_k = lax.optimization_barrier(w_k2).reshape(-1)
        part = ck(xp, w_k, b16)
        total = part if total is None else total + part
    return total.reshape(BATCH, 1)


# speedup vs baseline: 2.3806x; 1.0080x over previous
import functools

import jax
import jax.numpy as jnp
from jax import lax
from jax.experimental import pallas as pl
from jax.experimental.pallas import tpu as pltpu
from jax.experimental.pallas import tpu_sc as plsc

NUM_FIELDS = 26
BATCH = 16384
FIELD_SIZE = 100000
TOTAL_ROWS = NUM_FIELDS * FIELD_SIZE
LANES = 16
NUM_WORKERS = 32
BPW = BATCH // NUM_WORKERS
JV = BPW // LANES
N_EL = NUM_FIELDS * BPW

CHUNK_FIELDS = (2, 12, 12)


def _round_up(n, m):
    return (n + m - 1) // m * m


_CHUNK_SPECS = []
_f0 = 0
for _nf in CHUNK_FIELDS:
    _lo, _hi = _f0 * FIELD_SIZE, (_f0 + _nf) * FIELD_SIZE
    _size = _round_up(_hi - _lo, 1024)
    _start = min(_lo, TOTAL_ROWS - _size)
    _CHUNK_SPECS.append((_f0, _nf, _start, _size, _lo - _start))
    _f0 += _nf

_mesh = plsc.VectorSubcoreMesh(core_axis_name="c", subcore_axis_name="s")


def _make_chunk_kernel(f0, nf, size, local_base, add_bias):
    n_el = nf * BPW

    @functools.partial(
        pl.kernel,
        mesh=_mesh,
        out_type=jax.ShapeDtypeStruct((BATCH,), jnp.float32),
        scratch_types=[
            pltpu.VMEM((n_el,), jnp.int32),
            pltpu.VMEM((n_el,), jnp.float32),
            pltpu.VMEM((LANES,), jnp.float32),
            pltpu.VMEM((BPW,), jnp.float32),
            pltpu.SemaphoreType.DMA,
        ],
    )
    def _chunk_kernel(x_hbm, w_hbm, b_hbm, out_hbm, xv, gath, bias_v, out_v,
                      sem):
        wid = lax.axis_index("s") * 2 + lax.axis_index("c")
        base = wid * BPW
        pltpu.sync_copy(x_hbm.at[pl.ds(wid * N_EL + f0 * BPW, n_el)], xv)
        if add_bias:
            pltpu.sync_copy(b_hbm, bias_v)

        def off_body(v, _):
            off = local_base + (v // JV) * FIELD_SIZE
            xv[pl.ds(v * LANES, LANES)] = xv[pl.ds(v * LANES, LANES)] + off
            return 0

        lax.fori_loop(0, nf * JV, off_body, 0)

        pltpu.async_copy(w_hbm.at[xv], gath, sem).wait()

        if add_bias:
            seed = bias_v[...]
        else:
            seed = jnp.zeros((LANES,), jnp.float32)

        def r_body(j, _):
            def a_body(f, acc):
                return acc + gath[pl.ds(f * BPW + j * LANES, LANES)]

            out_v[pl.ds(j * LANES, LANES)] = lax.fori_loop(0, nf, a_body,
                                                           seed)
            return 0

        lax.fori_loop(0, JV, r_body, 0)
        pltpu.sync_copy(out_v, out_hbm.at[pl.ds(base, BPW)])

    return _chunk_kernel


_CHUNK_KERNELS = [
    _make_chunk_kernel(f0, nf, size, local_base, f0 == 0)
    for f0, nf, start, size, local_base in _CHUNK_SPECS
]


def kernel(x, fc_weight, bias):
    xp = (x.astype(jnp.int32).T
          .reshape(NUM_FIELDS, NUM_WORKERS, BPW)
          .transpose(1, 0, 2)
          .reshape(-1))
    b16 = jnp.broadcast_to(bias.astype(jnp.float32), (LANES,))
    total = None
    for (f0, nf, start, size, local_base), ck in zip(_CHUNK_SPECS,
                                                     _CHUNK_KERNELS):
        w_k2 = lax.slice(fc_weight, (start, 0), (start + size, 1))
        w_k = lax.optimization_barrier(w_k2).reshape(-1)
        part = ck(xp, w_k, b16)
        total = part if total is None else total + part
    return total.reshape(BATCH, 1)


# speedup vs baseline: 3.0637x; 1.2869x over previous
import functools

import jax
import jax.numpy as jnp
from jax import lax
from jax.experimental import pallas as pl
from jax.experimental.pallas import tpu as pltpu
from jax.experimental.pallas import tpu_sc as plsc

NUM_FIELDS = 26
BATCH = 16384
FIELD_SIZE = 100000
TOTAL_ROWS = NUM_FIELDS * FIELD_SIZE
LANES = 16
NUM_WORKERS = 32
BPW = BATCH // NUM_WORKERS
JV = BPW // LANES
N_EL = NUM_FIELDS * BPW

CALL_SUBS = (((0, 6), (6, 6)), ((12, 6), (18, 8)))


def _round_up(n, m):
    return (n + m - 1) // m * m


def _slice_spec(f0, nf):
    lo, hi = f0 * FIELD_SIZE, (f0 + nf) * FIELD_SIZE
    size = _round_up(hi - lo, 1024)
    start = min(lo, TOTAL_ROWS - size)
    return start, size, lo - start

_mesh = plsc.VectorSubcoreMesh(core_axis_name="c", subcore_axis_name="s")


def _make_call_kernel(subs, add_bias):
    specs = [(f0, nf) + _slice_spec(f0, nf) for f0, nf in subs]
    n_els = [nf * BPW for _, nf, _, _, _ in specs]

    scratch = []
    for n_el in n_els:
        scratch.append(pltpu.VMEM((n_el,), jnp.int32))
        scratch.append(pltpu.VMEM((n_el,), jnp.float32))
    scratch += [
        pltpu.VMEM((LANES,), jnp.float32),
        pltpu.VMEM((BPW,), jnp.float32),
        pltpu.SemaphoreType.DMA,
    ]

    @functools.partial(
        pl.kernel,
        mesh=_mesh,
        out_type=jax.ShapeDtypeStruct((BATCH,), jnp.float32),
        scratch_types=scratch,
    )
    def _call_kernel(x_hbm, *args):
        w_hbms = args[:len(specs)]
        b_hbm = args[len(specs)]
        out_hbm = args[len(specs) + 1]
        rest = args[len(specs) + 2:]
        xvs = rest[0:2 * len(specs):2]
        gaths = rest[1:2 * len(specs):2]
        bias_v = rest[2 * len(specs)]
        out_v = rest[2 * len(specs) + 1]
        sem = rest[2 * len(specs) + 2]

        wid = lax.axis_index("s") * 2 + lax.axis_index("c")
        base = wid * BPW

        for (f0, nf, _, _, _), xv, n_el in zip(specs, xvs, n_els):
            pltpu.sync_copy(x_hbm.at[pl.ds(wid * N_EL + f0 * BPW, n_el)], xv)
        if add_bias:
            pltpu.sync_copy(b_hbm, bias_v)

        copies = []
        for (f0, nf, _, _, local_base), xv, gath, w_hbm in zip(
                specs, xvs, gaths, w_hbms):
            def off_body(v, _, xv=xv, local_base=local_base):
                off = local_base + (v // JV) * FIELD_SIZE
                xv[pl.ds(v * LANES, LANES)] = (xv[pl.ds(v * LANES, LANES)]
                                               + off)
                return 0

            lax.fori_loop(0, nf * JV, off_body, 0)
            copies.append(pltpu.async_copy(w_hbm.at[xv], gath, sem))
        for c in copies:
            c.wait()

        if add_bias:
            seed = bias_v[...]
        else:
            seed = jnp.zeros((LANES,), jnp.float32)

        def r_body(j, _):
            acc = seed
            for (f0, nf, _, _, _), gath in zip(specs, gaths):
                def a_body(f, acc, gath=gath, j=j):
                    return acc + gath[pl.ds(f * BPW + j * LANES, LANES)]

                acc = lax.fori_loop(0, nf, a_body, acc)
            out_v[pl.ds(j * LANES, LANES)] = acc
            return 0

        lax.fori_loop(0, JV, r_body, 0)
        pltpu.sync_copy(out_v, out_hbm.at[pl.ds(base, BPW)])

    return _call_kernel


_CALL_KERNELS = [_make_call_kernel(subs, i == 0)
                 for i, subs in enumerate(CALL_SUBS)]


def kernel(x, fc_weight, bias):
    xp = (x.astype(jnp.int32).T
          .reshape(NUM_FIELDS, NUM_WORKERS, BPW)
          .transpose(1, 0, 2)
          .reshape(-1))
    b16 = jnp.broadcast_to(bias.astype(jnp.float32), (LANES,))

    total = None
    prev_slices = ()
    for subs, ck in zip(CALL_SUBS, _CALL_KERNELS):
        sl2d = []
        for f0, nf in subs:
            start, size, _ = _slice_spec(f0, nf)
            sl2d.append(lax.slice(fc_weight, (start, 0), (start + size, 1)))
        bar = lax.optimization_barrier(tuple(sl2d) + prev_slices)
        ws = [b.reshape(-1) for b in bar[:len(sl2d)]]
        prev_slices = tuple(bar[:len(sl2d)])
        part = ck(xp, *ws, b16)
        total = part if total is None else total + part
    return total.reshape(BATCH, 1)


# speedup vs baseline: 3.0809x; 1.0056x over previous
import functools

import jax
import jax.numpy as jnp
from jax import lax
from jax.experimental import pallas as pl
from jax.experimental.pallas import tpu as pltpu
from jax.experimental.pallas import tpu_sc as plsc

NUM_FIELDS = 26
BATCH = 16384
FIELD_SIZE = 100000
TOTAL_ROWS = NUM_FIELDS * FIELD_SIZE
LANES = 16
NUM_WORKERS = 32
BPW = BATCH // NUM_WORKERS
JV = BPW // LANES
N_EL = NUM_FIELDS * BPW

CALL_SUBS = (((0, 6), (6, 6)), ((12, 6), (18, 8)))


def _round_up(n, m):
    return (n + m - 1) // m * m


def _slice_spec(f0, nf):
    lo, hi = f0 * FIELD_SIZE, (f0 + nf) * FIELD_SIZE
    size = _round_up(hi - lo, 1024)
    start = min(lo, TOTAL_ROWS - size)
    return start, size, lo - start

_mesh = plsc.VectorSubcoreMesh(core_axis_name="c", subcore_axis_name="s")


def _make_call_kernel(subs, add_bias):
    specs = [(f0, nf) + _slice_spec(f0, nf) for f0, nf in subs]
    n_els = [nf * BPW for _, nf, _, _, _ in specs]

    scratch = []
    for n_el in n_els:
        scratch.append(pltpu.VMEM((n_el,), jnp.int32))
        scratch.append(pltpu.VMEM((n_el,), jnp.float32))
    scratch += [
        pltpu.VMEM((LANES,), jnp.float32),
        pltpu.VMEM((BPW,), jnp.float32),
        pltpu.SemaphoreType.DMA,
        pltpu.SemaphoreType.DMA,
    ]

    @functools.partial(
        pl.kernel,
        mesh=_mesh,
        out_type=jax.ShapeDtypeStruct((BATCH,), jnp.float32),
        scratch_types=scratch,
    )
    def _call_kernel(x_hbm, *args):
        ns = len(specs)
        w_hbms = args[:ns]
        b_hbm = args[ns]
        out_hbm = args[ns + 1]
        rest = args[ns + 2:]
        xvs = rest[0:2 * ns:2]
        gaths = rest[1:2 * ns:2]
        bias_v = rest[2 * ns]
        out_v = rest[2 * ns + 1]
        sem_x = rest[2 * ns + 2]
        sem_g = rest[2 * ns + 3]

        wid = lax.axis_index("s") * 2 + lax.axis_index("c")
        base = wid * BPW

        x_copies = [
            pltpu.async_copy(
                x_hbm.at[pl.ds(wid * N_EL + f0 * BPW, n_el)], xv, sem_x)
            for (f0, _, _, _, _), xv, n_el in zip(specs, xvs, n_els)
        ]
        if add_bias:
            pltpu.sync_copy(b_hbm, bias_v)

        g_copies = []
        for (f0, nf, _, _, local_base), xv, gath, w_hbm, xc in zip(
                specs, xvs, gaths, w_hbms, x_copies):
            xc.wait()

            def off_body(v, _, xv=xv, local_base=local_base):
                off = local_base + (v // JV) * FIELD_SIZE
                xv[pl.ds(v * LANES, LANES)] = (xv[pl.ds(v * LANES, LANES)]
                                               + off)
                return 0

            lax.fori_loop(0, nf * JV, off_body, 0)
            g_copies.append(pltpu.async_copy(w_hbm.at[xv], gath, sem_g))

        if add_bias:
            seed = bias_v[...]
        else:
            seed = jnp.zeros((LANES,), jnp.float32)

        for si, ((f0, nf, _, _, _), gath, gc) in enumerate(
                zip(specs, gaths, g_copies)):
            gc.wait()
            first = si == 0

            def r_body(j, _, gath=gath, nf=nf, first=first):
                def a_body(f, acc, gath=gath, j=j):
                    return acc + gath[pl.ds(f * BPW + j * LANES, LANES)]

                init = seed if first else out_v[pl.ds(j * LANES, LANES)]
                out_v[pl.ds(j * LANES, LANES)] = lax.fori_loop(0, nf, a_body,
                                                               init)
                return 0

            lax.fori_loop(0, JV, r_body, 0)

        pltpu.sync_copy(out_v, out_hbm.at[pl.ds(base, BPW)])

    return _call_kernel


_CALL_KERNELS = [_make_call_kernel(subs, i == 0)
                 for i, subs in enumerate(CALL_SUBS)]


def kernel(x, fc_weight, bias):
    xp = (x.astype(jnp.int32)
          .reshape(NUM_WORKERS, BPW, NUM_FIELDS)
          .transpose(0, 2, 1)
          .reshape(-1))
    b16 = jnp.broadcast_to(bias.astype(jnp.float32), (LANES,))

    total = None
    dep = None
    for subs, ck in zip(CALL_SUBS, _CALL_KERNELS):
        sl2d = []
        for f0, nf in subs:
            start, size, _ = _slice_spec(f0, nf)
            s = lax.slice(fc_weight, (start, 0), (start + size, 1))
            if dep is not None:
                s = s + dep
            sl2d.append(s)
        bar = lax.optimization_barrier(tuple(sl2d))
        ws = [b.reshape(-1) for b in bar]
        dep = bar[0][:1, :] * 0.0
        part = ck(xp, *ws, b16)
        total = part if total is None else total + part
    return total.reshape(BATCH, 1)
